# baseline (speedup 1.0000x reference)
import jax
import jax.numpy as jnp
from jax import lax
from jax.experimental import pallas as pl
from jax.experimental.pallas import tpu as pltpu
from jax.experimental.pallas import tpu_sc as plsc

NUM_CLASSES = 1000000
EMBED = 64
BATCH = 4096
SEQ = 200

NC = 2
NS = 16
NW = NC * NS

BW = 128
LC = 8
NCH = SEQ // LC
SUB = 256
NSUB = LC * BW // SUB
EH = EMBED // 2


def _body(x_hbm, tab_hbm, out_hbm, xsl, idxT, rows, locA, locB,
          sg0, sg1, ssA, ssB):
    wid = lax.axis_index("s") * NC + lax.axis_index("c")
    b0 = wid * BW
    sg = (sg0, sg1)

    eye = lax.iota(jnp.int32, 16)

    def start_gather(s, p):
        for c in range(SUB // 128):
            pltpu.make_async_copy(
                tab_hbm.at[idxT.at[pl.ds(s * SUB + c * 128, 128)]],
                rows.at[p].at[pl.ds(c * 128, 128)],
                sg[p],
            ).start()

    def wait_gather(p):
        for c in range(SUB // 128):
            pltpu.make_async_copy(
                tab_hbm.at[pl.ds(0, 128)],
                rows.at[p].at[pl.ds(c * 128, 128)],
                sg[p],
            ).wait()

    def start_stores(i):
        pltpu.make_async_copy(
            locA, out_hbm.at[pl.ds(0, EH), i, wid], ssA
        ).start()
        pltpu.make_async_copy(
            locB, out_hbm.at[pl.ds(EH, EH), i, wid], ssB
        ).start()

    def wait_stores(i):
        pltpu.make_async_copy(
            locA, out_hbm.at[pl.ds(0, EH), i, wid], ssA
        ).wait()
        pltpu.make_async_copy(
            locB, out_hbm.at[pl.ds(EH, EH), i, wid], ssB
        ).wait()

    pltpu.sync_copy(x_hbm.at[pl.ds(b0, BW)], xsl)

    def chunk(i, carry):
        for lp in range(LC):
            lcol = jnp.full((16,), i * LC + lp, jnp.int32)
            for bb in range(BW // 16):
                v = plsc.load_gather(xsl, [eye + bb * 16, lcol])
                idxT[pl.ds(lp * BW + bb * 16, 16)] = v

        @pl.when(i >= 1)
        def _():
            wait_stores(i - 1)

        start_gather(0, 0)
        start_gather(1, 1)
        for s in range(NSUB):
            p = s % 2
            wait_gather(p)

            @plsc.parallel_loop(0, SUB, step=1, unroll=4)
            def _(k):
                lp = jnp.full((16,), (s * SUB + k) // BW, jnp.int32)
                bj = jnp.full((16,), k & (BW - 1), jnp.int32)
                for eb in range(EMBED // 16):
                    v = rows.at[p][k, pl.ds(eb * 16, 16)]
                    if eb < 2:
                        plsc.store_scatter(
                            locA, [eye + eb * 16, lp, bj], v)
                    else:
                        plsc.store_scatter(
                            locB, [eye + (eb - 2) * 16, lp, bj], v)

            if s + 2 < NSUB:
                start_gather(s + 2, p)

        start_stores(i)
        return carry

    lax.fori_loop(0, NCH, chunk, 0)
    wait_stores(NCH - 1)


@jax.jit
def _run(x, embed_table):
    f = pl.kernel(
        _body,
        out_type=jax.ShapeDtypeStruct((EMBED, NCH, BATCH // BW, LC, BW),
                                      jnp.float32),
        mesh=plsc.VectorSubcoreMesh(
            core_axis_name="c", subcore_axis_name="s",
            num_cores=NC, num_subcores=NS,
        ),
        scratch_types=[
            pltpu.VMEM((BW, SEQ), jnp.int32),
            pltpu.VMEM((LC * BW,), jnp.int32),
            pltpu.VMEM((2, SUB, EMBED), jnp.float32),
            pltpu.VMEM((EH, LC, BW), jnp.float32),
            pltpu.VMEM((EH, LC, BW), jnp.float32),
            pltpu.SemaphoreType.DMA,
            pltpu.SemaphoreType.DMA,
            pltpu.SemaphoreType.DMA,
            pltpu.SemaphoreType.DMA,
        ],
        compiler_params=pltpu.CompilerParams(
            use_tc_tiling_on_sc=False, needs_layout_passes=False
        ),
    )
    out5 = f(x, embed_table)
    return out5.transpose(2, 4, 0, 1, 3).reshape(BATCH, EMBED, SEQ)


def kernel(x, embed_table):
    return _run(x, embed_table)


# speedup vs baseline: 1.0016x; 1.0016x over previous
import jax
import jax.numpy as jnp
from jax import lax
from jax.experimental import pallas as pl
from jax.experimental.pallas import tpu as pltpu
from jax.experimental.pallas import tpu_sc as plsc

NUM_CLASSES = 1000000
EMBED = 64
BATCH = 4096
SEQ = 200

NC = 2
NS = 16
NW = NC * NS

BW = 128
LC = 8
NCH = SEQ // LC
SUB = 256
NSUB = LC * BW // SUB
EH = EMBED // 2


def _body(x_hbm, tab_hbm, out_hbm, xsl, idxT, rows, locA, locB,
          sg0, sg1, ssA, ssB):
    wid = lax.axis_index("s") * NC + lax.axis_index("c")
    b0 = wid * BW
    sg = (sg0, sg1)

    eye = lax.iota(jnp.int32, 16)

    def start_gather(s, p):
        for c in range(SUB // 128):
            pltpu.make_async_copy(
                tab_hbm.at[idxT.at[pl.ds(s * SUB + c * 128, 128)]],
                rows.at[p].at[pl.ds(c * 128, 128)],
                sg[p],
            ).start()

    def wait_gather(p):
        for c in range(SUB // 128):
            pltpu.make_async_copy(
                tab_hbm.at[pl.ds(0, 128)],
                rows.at[p].at[pl.ds(c * 128, 128)],
                sg[p],
            ).wait()

    def start_stores(i):
        pltpu.make_async_copy(
            locA, out_hbm.at[pl.ds(0, EH), i, wid], ssA
        ).start()
        pltpu.make_async_copy(
            locB, out_hbm.at[pl.ds(EH, EH), i, wid], ssB
        ).start()

    def wait_stores(i):
        pltpu.make_async_copy(
            locA, out_hbm.at[pl.ds(0, EH), i, wid], ssA
        ).wait()
        pltpu.make_async_copy(
            locB, out_hbm.at[pl.ds(EH, EH), i, wid], ssB
        ).wait()

    pltpu.sync_copy(x_hbm.at[pl.ds(b0, BW)], xsl)

    def chunk(i, carry):
        for lp in range(LC):
            lcol = jnp.full((16,), i * LC + lp, jnp.int32)
            for bb in range(BW // 16):
                v = plsc.load_gather(xsl, [eye + bb * 16, lcol])
                idxT[pl.ds(lp * BW + bb * 16, 16)] = v

        @pl.when(i >= 1)
        def _():
            wait_stores(i - 1)

        start_gather(0, 0)
        start_gather(1, 1)
        for s in range(NSUB):
            p = s % 2
            wait_gather(p)

            @plsc.parallel_loop(0, SUB, step=1, unroll=4)
            def _(k):
                col = jnp.full((16,), s * SUB + k, jnp.int32)
                for eb in range(EMBED // 16):
                    v = rows.at[p][k, pl.ds(eb * 16, 16)]
                    if eb < 2:
                        plsc.store_scatter(locA, [eye + eb * 16, col], v)
                    else:
                        plsc.store_scatter(
                            locB, [eye + (eb - 2) * 16, col], v)

            if s + 2 < NSUB:
                start_gather(s + 2, p)

        start_stores(i)
        return carry

    lax.fori_loop(0, NCH, chunk, 0)
    wait_stores(NCH - 1)


@jax.jit
def _run(x, embed_table):
    f = pl.kernel(
        _body,
        out_type=jax.ShapeDtypeStruct((EMBED, NCH, BATCH // BW, LC * BW),
                                      jnp.float32),
        mesh=plsc.VectorSubcoreMesh(
            core_axis_name="c", subcore_axis_name="s",
            num_cores=NC, num_subcores=NS,
        ),
        scratch_types=[
            pltpu.VMEM((BW, SEQ), jnp.int32),
            pltpu.VMEM((LC * BW,), jnp.int32),
            pltpu.VMEM((2, SUB, EMBED), jnp.float32),
            pltpu.VMEM((EH, LC * BW), jnp.float32),
            pltpu.VMEM((EH, LC * BW), jnp.float32),
            pltpu.SemaphoreType.DMA,
            pltpu.SemaphoreType.DMA,
            pltpu.SemaphoreType.DMA,
            pltpu.SemaphoreType.DMA,
        ],
        compiler_params=pltpu.CompilerParams(
            use_tc_tiling_on_sc=False, needs_layout_passes=False
        ),
    )
    out5 = f(x, embed_table)
    out5 = out5.reshape(EMBED, NCH, BATCH // BW, LC, BW)
    return out5.transpose(2, 4, 0, 1, 3).reshape(BATCH, EMBED, SEQ)


def kernel(x, embed_table):
    return _run(x, embed_table)


# speedup vs baseline: 1.0740x; 1.0723x over previous
import jax
import jax.numpy as jnp
from jax import lax
from jax.experimental import pallas as pl
from jax.experimental.pallas import tpu as pltpu
from jax.experimental.pallas import tpu_sc as plsc

NUM_CLASSES = 1000000
EMBED = 64
BATCH = 4096
SEQ = 200

NC = 2
NS = 16
NW = NC * NS
ROWS_PER_W = BATCH // NW

CHUNKS = ((0, 128), (128, 72))


NG = 4
NO = 2


def _body(x_hbm, tab_hbm, out_hbm, idx_all, rows4, out2, sg0, sg1, sg2, sg3,
          so0, so1):
    wid = lax.axis_index("s") * NC + lax.axis_index("c")
    row0 = wid * ROWS_PER_W
    sg = (sg0, sg1, sg2, sg3)
    so = (so0, so1)

    pltpu.sync_copy(x_hbm.at[pl.ds(row0 * SEQ, ROWS_PER_W * SEQ)], idx_all)

    def start_gather(r, p):
        base = r * SEQ
        for off, n in CHUNKS:
            pltpu.make_async_copy(
                tab_hbm.at[idx_all.at[pl.ds(base + off, n)]],
                rows4.at[p].at[pl.ds(off, n)],
                sg[p],
            ).start()

    def wait_gather(p):
        for off, n in CHUNKS:
            pltpu.make_async_copy(
                tab_hbm.at[pl.ds(0, n)],
                rows4.at[p].at[pl.ds(off, n)],
                sg[p],
            ).wait()

    eye = lax.iota(jnp.int32, 16)

    def transpose(rbuf, obuf):
        @plsc.parallel_loop(0, SEQ, step=1, unroll=4)
        def _(l):
            col = jnp.full((16,), l, jnp.int32)
            for eb in range(EMBED // 16):
                v = rbuf[l, pl.ds(eb * 16, 16)]
                plsc.store_scatter(obuf, [eye + (eb * 16), col], v)

    def start_store(r, q):
        pltpu.make_async_copy(out2.at[q], out_hbm.at[row0 + r], so[q]).start()

    def wait_store(r, q):
        pltpu.make_async_copy(out2.at[q], out_hbm.at[row0 + r], so[q]).wait()

    for r in range(NG - 1):
        start_gather(r, r)

    def step(k, carry):
        for j in range(NG):
            r = NG * k + j
            p = j
            q = j % NO

            @pl.when(r + (NG - 1) < ROWS_PER_W)
            def _():
                start_gather(r + (NG - 1), (j + NG - 1) % NG)

            wait_gather(p)

            @pl.when(r >= NO)
            def _():
                wait_store(r - NO, q)

            transpose(rows4.at[p], out2.at[q])
            start_store(r, q)
        return carry

    lax.fori_loop(0, ROWS_PER_W // NG, step, 0)
    wait_store(ROWS_PER_W - 2, 0)
    wait_store(ROWS_PER_W - 1, 1)


@jax.jit
def _run(x, embed_table):
    mesh = plsc.VectorSubcoreMesh(
        core_axis_name="c", subcore_axis_name="s", num_cores=NC, num_subcores=NS
    )
    f = pl.kernel(
        _body,
        out_type=jax.ShapeDtypeStruct((BATCH, EMBED, SEQ), jnp.float32),
        mesh=mesh,
        scratch_types=[
            pltpu.VMEM((ROWS_PER_W * SEQ,), jnp.int32),
            pltpu.VMEM((NG, SEQ, EMBED), jnp.float32),
            pltpu.VMEM((NO, EMBED, SEQ), jnp.float32),
            pltpu.SemaphoreType.DMA,
            pltpu.SemaphoreType.DMA,
            pltpu.SemaphoreType.DMA,
            pltpu.SemaphoreType.DMA,
            pltpu.SemaphoreType.DMA,
            pltpu.SemaphoreType.DMA,
        ],
        compiler_params=pltpu.CompilerParams(
            use_tc_tiling_on_sc=False, needs_layout_passes=False
        ),
    )
    return f(x.reshape(-1), embed_table)


def kernel(x, embed_table):
    return _run(x, embed_table)
